# baseline (speedup 1.0000x reference)
import functools

import jax
import jax.numpy as jnp
from jax import lax
from jax.experimental import pallas as pl
from jax.experimental.pallas import tpu as pltpu
from jax.experimental.pallas import tpu_sc as plsc

_VOCAB = 100000
_EMB = 64
_RMAX = 7
_B = 1024
_L = 200
_NTOK = _B * _L

_WPAD = 128
_KPAD = 512

_NC, _NS = 2, 16
_NW = _NC * _NS
_TOK_PER_W = _NTOK // _NW
_CHUNK = 64
_NBUF = 2
_NCHUNK = _TOK_PER_W // _CHUNK
_NROUND = _NCHUNK // _NBUF


def _sc_gather(seq_flat, Wp, Kp):
    mesh = plsc.VectorSubcoreMesh(core_axis_name="c", subcore_axis_name="s")

    @functools.partial(
        pl.kernel,
        out_type=(
            jax.ShapeDtypeStruct((_NTOK, _WPAD), jnp.float32),
            jax.ShapeDtypeStruct((_NTOK, _KPAD), jnp.float32),
        ),
        mesh=mesh,
        scratch_types=[
            pltpu.VMEM((_NBUF, _CHUNK), jnp.int32),
            pltpu.VMEM((_NBUF, _CHUNK, _WPAD), jnp.float32),
            pltpu.VMEM((_NBUF, _CHUNK, _KPAD), jnp.float32),
            pltpu.SemaphoreType.DMA,
            pltpu.SemaphoreType.DMA,
            pltpu.SemaphoreType.DMA,
            pltpu.SemaphoreType.DMA,
        ],
    )
    def gather_kernel(seq_hbm, w_hbm, k_hbm, wg_hbm, kg_hbm,
                      idx_v, wrow_v, krow_v, gsem0, gsem1, wsem0, wsem1):
        wid = lax.axis_index("s") * _NC + lax.axis_index("c")
        base = wid * _TOK_PER_W
        gsems = (gsem0, gsem1)
        wsems = (wsem0, wsem1)

        def wait_writeback(b):
            pltpu.make_async_copy(
                wrow_v.at[b], wg_hbm.at[pl.ds(base, _CHUNK)], wsems[b]).wait()
            pltpu.make_async_copy(
                krow_v.at[b], kg_hbm.at[pl.ds(base, _CHUNK)], wsems[b]).wait()

        @pl.loop(0, _NROUND)
        def _round(g):
            @pl.when(g > 0)
            def _():
                wait_writeback(0)
                wait_writeback(1)

            copies = []
            for b in range(_NBUF):
                off = base + (g * _NBUF + b) * _CHUNK
                pltpu.sync_copy(seq_hbm.at[pl.ds(off, _CHUNK)], idx_v.at[b])
                cw = pltpu.async_copy(w_hbm.at[idx_v.at[b]], wrow_v.at[b],
                                      gsems[b])
                ck = pltpu.async_copy(k_hbm.at[idx_v.at[b]], krow_v.at[b],
                                      gsems[b])
                copies.append((cw, ck))
            for b in range(_NBUF):
                cw, ck = copies[b]
                cw.wait()
                ck.wait()
                off = base + (g * _NBUF + b) * _CHUNK
                pltpu.async_copy(wrow_v.at[b], wg_hbm.at[pl.ds(off, _CHUNK)],
                                 wsems[b])
                pltpu.async_copy(krow_v.at[b], kg_hbm.at[pl.ds(off, _CHUNK)],
                                 wsems[b])

        wait_writeback(0)
        wait_writeback(1)

    return gather_kernel(seq_flat, Wp, Kp)


def _tc_merge(Wg, Kg):
    bb = 8
    n3, n5, n7 = _L - 2, _L - 4, _L - 6

    def body(wg_ref, kg_ref, o3_ref, o5_ref, o7_ref):
        for b in range(bb):
            def prod(d, clo, n):
                w = wg_ref[b, pl.ds(clo + d, n), pl.ds(0, _EMB)]
                k = kg_ref[b, pl.ds(clo, n), pl.ds(_EMB * (3 + d), _EMB)]
                return w * k

            m = prod(-1, 1, n3)
            m = jnp.maximum(m, prod(0, 1, n3))
            m = jnp.maximum(m, prod(1, 1, n3))
            o3_ref[b] = m
            m = m[1:1 + n5]
            m = jnp.maximum(m, prod(-2, 2, n5))
            m = jnp.maximum(m, prod(2, 2, n5))
            o5_ref[b] = m
            m = m[1:1 + n7]
            m = jnp.maximum(m, prod(-3, 3, n7))
            m = jnp.maximum(m, prod(3, 3, n7))
            o7_ref[b] = m

    out = pl.pallas_call(
        body,
        grid=(_B // bb,),
        in_specs=[
            pl.BlockSpec((bb, _L, _WPAD), lambda i: (i, 0, 0)),
            pl.BlockSpec((bb, _L, _KPAD), lambda i: (i, 0, 0)),
        ],
        out_specs=[
            pl.BlockSpec((bb, n3, _EMB), lambda i: (i, 0, 0)),
            pl.BlockSpec((bb, n5, _EMB), lambda i: (i, 0, 0)),
            pl.BlockSpec((bb, n7, _EMB), lambda i: (i, 0, 0)),
        ],
        out_shape=[
            jax.ShapeDtypeStruct((_B, n3, _EMB), jnp.float32),
            jax.ShapeDtypeStruct((_B, n5, _EMB), jnp.float32),
            jax.ShapeDtypeStruct((_B, n7, _EMB), jnp.float32),
        ],
    )(Wg, Kg)
    return tuple(out)


@jax.jit
def kernel(seq, W, K):
    seq_flat = seq.astype(jnp.int32).reshape(-1)
    K2 = K.reshape(_VOCAB, _RMAX * _EMB)
    Wp = jnp.concatenate([W, W], axis=1)
    Kp = jnp.concatenate([K2, K2[:, :_KPAD - _RMAX * _EMB]], axis=1)
    Wg, Kg = _sc_gather(seq_flat, Wp, Kp)
    Wg = Wg.reshape(_B, _L, _WPAD)
    Kg = Kg.reshape(_B, _L, _KPAD)
    return _tc_merge(Wg, Kg)


# speedup vs baseline: 1.5219x; 1.5219x over previous
import functools

import jax
import jax.numpy as jnp
from jax import lax
from jax.experimental import pallas as pl
from jax.experimental.pallas import tpu as pltpu
from jax.experimental.pallas import tpu_sc as plsc

_VOCAB = 100000
_EMB = 64
_RMAX = 7
_B = 1024
_L = 200
_NTOK = _B * _L

_WPAD = 128
_KPAD = 512

_NC, _NS = 2, 16
_NW = _NC * _NS
_TOK_PER_W = _NTOK // _NW
_CHUNK = 64
_NBUF = 2
_NCHUNK = _TOK_PER_W // _CHUNK
_NROUND = _NCHUNK // _NBUF


def _sc_gather(seq_flat, Wp, Kp):
    mesh = plsc.VectorSubcoreMesh(core_axis_name="c", subcore_axis_name="s")

    @functools.partial(
        pl.kernel,
        out_type=(
            jax.ShapeDtypeStruct((_NTOK, _WPAD), jnp.float32),
            jax.ShapeDtypeStruct((_NTOK, _KPAD), jnp.float32),
        ),
        mesh=mesh,
        scratch_types=[
            pltpu.VMEM((_NBUF, _CHUNK), jnp.int32),
            pltpu.VMEM((_NBUF, _CHUNK, _WPAD), jnp.float32),
            pltpu.VMEM((_NBUF, _CHUNK, _KPAD), jnp.float32),
            pltpu.SemaphoreType.DMA,
            pltpu.SemaphoreType.DMA,
            pltpu.SemaphoreType.DMA,
            pltpu.SemaphoreType.DMA,
        ],
    )
    def gather_kernel(seq_hbm, w_hbm, k_hbm, wg_hbm, kg_hbm,
                      idx_v, wrow_v, krow_v, gsem0, gsem1, wsem0, wsem1):
        wid = lax.axis_index("s") * _NC + lax.axis_index("c")
        base = wid * _TOK_PER_W
        gsems = (gsem0, gsem1)
        wsems = (wsem0, wsem1)

        def wait_writeback(b):
            pltpu.make_async_copy(
                wrow_v.at[b], wg_hbm.at[pl.ds(base, _CHUNK)], wsems[b]).wait()
            pltpu.make_async_copy(
                krow_v.at[b], kg_hbm.at[pl.ds(base, _CHUNK)], wsems[b]).wait()

        @pl.loop(0, _NROUND)
        def _round(g):
            @pl.when(g > 0)
            def _():
                wait_writeback(0)
                wait_writeback(1)

            copies = []
            for b in range(_NBUF):
                off = base + (g * _NBUF + b) * _CHUNK
                pltpu.sync_copy(seq_hbm.at[pl.ds(off, _CHUNK)], idx_v.at[b])
                cw = pltpu.async_copy(w_hbm.at[idx_v.at[b]], wrow_v.at[b],
                                      gsems[b])
                ck = pltpu.async_copy(k_hbm.at[idx_v.at[b]], krow_v.at[b],
                                      gsems[b])
                copies.append((cw, ck))
            for b in range(_NBUF):
                cw, ck = copies[b]
                cw.wait()
                ck.wait()
                off = base + (g * _NBUF + b) * _CHUNK
                pltpu.async_copy(wrow_v.at[b], wg_hbm.at[pl.ds(off, _CHUNK)],
                                 wsems[b])
                pltpu.async_copy(krow_v.at[b], kg_hbm.at[pl.ds(off, _CHUNK)],
                                 wsems[b])

        wait_writeback(0)
        wait_writeback(1)

    return gather_kernel(seq_flat, Wp, Kp)


def _tc_pack_k(KT):
    vb = 2048
    nblk = (_VOCAB + vb - 1) // vb
    main_w = (_VOCAB - (nblk - 1) * vb) // 128 * 128
    vpad = nblk * vb

    def body(kt_hbm, ktail_ref, kp_ref, kt_v, sem):
        i = pl.program_id(0)

        def chunk(src_ref, lane0, s):
            cols = [src_ref[pl.ds(64 * j, 64), pl.ds(lane0, 128)].T
                    for j in range(_RMAX)]
            groups = [jnp.concatenate(cols[0:2], axis=1),
                      jnp.concatenate(cols[2:4], axis=1),
                      jnp.concatenate(cols[4:6], axis=1),
                      cols[6]]
            for g, val in enumerate(groups):
                kp_ref[pl.ds(128 * s, 128), pl.ds(128 * g, val.shape[1])] = val

        @pl.when(i < nblk - 1)
        def _():
            cp = pltpu.make_async_copy(
                kt_hbm.at[:, pl.ds(i * vb, vb)], kt_v, sem)
            cp.start()
            cp.wait()
            for s in range(vb // 128):
                chunk(kt_v, 128 * s, s)

        @pl.when(i == nblk - 1)
        def _():
            cp = pltpu.make_async_copy(
                kt_hbm.at[:, pl.ds(i * vb, main_w)],
                kt_v.at[:, pl.ds(0, main_w)], sem)
            cp.start()
            cp.wait()
            for s in range(main_w // 128):
                chunk(kt_v, 128 * s, s)
            chunk(ktail_ref, 0, main_w // 128)

    return pl.pallas_call(
        body,
        grid=(nblk,),
        in_specs=[pl.BlockSpec(memory_space=pl.ANY),
                  pl.BlockSpec((_RMAX * _EMB, 128), lambda i: (0, 0))],
        out_specs=pl.BlockSpec((vb, _KPAD), lambda i: (i, 0)),
        out_shape=jax.ShapeDtypeStruct((vpad, _KPAD), jnp.float32),
        scratch_shapes=[
            pltpu.VMEM((_RMAX * _EMB, vb), jnp.float32),
            pltpu.SemaphoreType.DMA,
        ],
    )(KT, jnp.pad(KT[:, (nblk - 1) * vb + main_w:],
                  ((0, 0), (0, 128 - (_VOCAB - (nblk - 1) * vb - main_w)))))


def _tc_merge(Wg, Kg):
    bb = 8
    n3, n5, n7 = _L - 2, _L - 4, _L - 6

    def body(wg_ref, kg_ref, o3_ref, o5_ref, o7_ref):
        for b in range(bb):
            def prod(d, clo, n):
                w = wg_ref[b, pl.ds(clo + d, n), pl.ds(0, _EMB)]
                k = kg_ref[b, pl.ds(clo, n), pl.ds(_EMB * (3 + d), _EMB)]
                return w * k

            m = prod(-1, 1, n3)
            m = jnp.maximum(m, prod(0, 1, n3))
            m = jnp.maximum(m, prod(1, 1, n3))
            o3_ref[b] = m
            m = m[1:1 + n5]
            m = jnp.maximum(m, prod(-2, 2, n5))
            m = jnp.maximum(m, prod(2, 2, n5))
            o5_ref[b] = m
            m = m[1:1 + n7]
            m = jnp.maximum(m, prod(-3, 3, n7))
            m = jnp.maximum(m, prod(3, 3, n7))
            o7_ref[b] = m

    out = pl.pallas_call(
        body,
        grid=(_B // bb,),
        in_specs=[
            pl.BlockSpec((bb, _L, _WPAD), lambda i: (i, 0, 0)),
            pl.BlockSpec((bb, _L, _KPAD), lambda i: (i, 0, 0)),
        ],
        out_specs=[
            pl.BlockSpec((bb, n3, _EMB), lambda i: (i, 0, 0)),
            pl.BlockSpec((bb, n5, _EMB), lambda i: (i, 0, 0)),
            pl.BlockSpec((bb, n7, _EMB), lambda i: (i, 0, 0)),
        ],
        out_shape=[
            jax.ShapeDtypeStruct((_B, n3, _EMB), jnp.float32),
            jax.ShapeDtypeStruct((_B, n5, _EMB), jnp.float32),
            jax.ShapeDtypeStruct((_B, n7, _EMB), jnp.float32),
        ],
    )(Wg, Kg)
    return tuple(out)


@jax.jit
def kernel(seq, W, K):
    seq_flat = seq.astype(jnp.int32).reshape(-1)
    Wp = jnp.concatenate([W, W], axis=1)
    KT = jnp.transpose(K, (1, 2, 0)).reshape(_RMAX * _EMB, _VOCAB)
    Kp = _tc_pack_k(KT)
    Wg, Kg = _sc_gather(seq_flat, Wp, Kp)
    Wg = Wg.reshape(_B, _L, _WPAD)
    Kg = Kg.reshape(_B, _L, _KPAD)
    return _tc_merge(Wg, Kg)


# speedup vs baseline: 1.7005x; 1.1174x over previous
import functools

import jax
import jax.numpy as jnp
from jax import lax
from jax.experimental import pallas as pl
from jax.experimental.pallas import tpu as pltpu
from jax.experimental.pallas import tpu_sc as plsc

_VOCAB = 100000
_EMB = 64
_RMAX = 7
_B = 1024
_L = 200
_NTOK = _B * _L

_WPAD = 128
_KPAD = 512

_NC, _NS = 2, 16
_NW = _NC * _NS
_TOK_PER_W = _NTOK // _NW
_NBUF = 2


def _sc_gather(seq_flat, table, row, chunk):
    mesh = plsc.VectorSubcoreMesh(core_axis_name="c", subcore_axis_name="s")
    nround = _TOK_PER_W // chunk // _NBUF

    @functools.partial(
        pl.kernel,
        out_type=jax.ShapeDtypeStruct((_NTOK, row), jnp.float32),
        mesh=mesh,
        scratch_types=[
            pltpu.VMEM((_NBUF, chunk), jnp.int32),
            pltpu.VMEM((_NBUF, chunk, row), jnp.float32),
            pltpu.SemaphoreType.DMA,
            pltpu.SemaphoreType.DMA,
            pltpu.SemaphoreType.DMA,
            pltpu.SemaphoreType.DMA,
        ],
    )
    def gather_kernel(seq_hbm, t_hbm, out_hbm,
                      idx_v, row_v, gsem0, gsem1, wsem0, wsem1):
        wid = lax.axis_index("s") * _NC + lax.axis_index("c")
        base = wid * _TOK_PER_W
        gsems = (gsem0, gsem1)
        wsems = (wsem0, wsem1)

        def wait_writeback(b):
            pltpu.make_async_copy(
                row_v.at[b], out_hbm.at[pl.ds(base, chunk)], wsems[b]).wait()

        @pl.loop(0, nround)
        def _round(g):
            @pl.when(g > 0)
            def _():
                wait_writeback(0)
                wait_writeback(1)

            copies = []
            for b in range(_NBUF):
                off = base + (g * _NBUF + b) * chunk
                pltpu.sync_copy(seq_hbm.at[pl.ds(off, chunk)], idx_v.at[b])
                copies.append(pltpu.async_copy(
                    t_hbm.at[idx_v.at[b]], row_v.at[b], gsems[b]))
            for b in range(_NBUF):
                copies[b].wait()
                off = base + (g * _NBUF + b) * chunk
                pltpu.async_copy(row_v.at[b], out_hbm.at[pl.ds(off, chunk)],
                                 wsems[b])

        wait_writeback(0)
        wait_writeback(1)

    return gather_kernel(seq_flat, table)


def _tc_pack_k(KT):
    vb = 2048
    nblk = (_VOCAB + vb - 1) // vb
    main_w = (_VOCAB - (nblk - 1) * vb) // 128 * 128
    vpad = nblk * vb

    def start_fetch(kt_hbm, kt_v, sems, i):
        b = lax.rem(i, 2)

        @pl.when(i < nblk - 1)
        def _():
            pltpu.make_async_copy(
                kt_hbm.at[:, pl.ds(i * vb, vb)], kt_v.at[b], sems.at[b]
            ).start()

        @pl.when(i == nblk - 1)
        def _():
            pltpu.make_async_copy(
                kt_hbm.at[:, pl.ds(i * vb, main_w)],
                kt_v.at[b, :, pl.ds(0, main_w)], sems.at[b]
            ).start()

    def body(kt_hbm, ktail_ref, kp_ref, kt_v, sems):
        i = pl.program_id(0)
        b = lax.rem(i, 2)

        @pl.when(i == 0)
        def _():
            start_fetch(kt_hbm, kt_v, sems, i)

        @pl.when(i < nblk - 1)
        def _():
            start_fetch(kt_hbm, kt_v, sems, i + 1)

        def chunk(src_ref, lane0, s):
            cols = [src_ref[pl.ds(64 * j, 64), pl.ds(lane0, 128)].T
                    for j in range(_RMAX)]
            groups = [jnp.concatenate(cols[0:2], axis=1),
                      jnp.concatenate(cols[2:4], axis=1),
                      jnp.concatenate(cols[4:6], axis=1),
                      cols[6]]
            for g, val in enumerate(groups):
                kp_ref[pl.ds(128 * s, 128), pl.ds(128 * g, val.shape[1])] = val

        @pl.when(i < nblk - 1)
        def _():
            pltpu.make_async_copy(
                kt_hbm.at[:, pl.ds(i * vb, vb)], kt_v.at[b], sems.at[b]
            ).wait()
            for s in range(vb // 128):
                chunk(kt_v.at[b], 128 * s, s)

        @pl.when(i == nblk - 1)
        def _():
            pltpu.make_async_copy(
                kt_hbm.at[:, pl.ds(i * vb, main_w)],
                kt_v.at[b, :, pl.ds(0, main_w)], sems.at[b]
            ).wait()
            for s in range(main_w // 128):
                chunk(kt_v.at[b], 128 * s, s)
            chunk(ktail_ref, 0, main_w // 128)

    return pl.pallas_call(
        body,
        grid=(nblk,),
        in_specs=[pl.BlockSpec(memory_space=pl.ANY),
                  pl.BlockSpec((_RMAX * _EMB, 128), lambda i: (0, 0))],
        out_specs=pl.BlockSpec((vb, _KPAD), lambda i: (i, 0)),
        out_shape=jax.ShapeDtypeStruct((vpad, _KPAD), jnp.float32),
        scratch_shapes=[
            pltpu.VMEM((2, _RMAX * _EMB, vb), jnp.float32),
            pltpu.SemaphoreType.DMA((2,)),
        ],
    )(KT, jnp.pad(KT[:, (nblk - 1) * vb + main_w:],
                  ((0, 0), (0, 128 - (_VOCAB - (nblk - 1) * vb - main_w)))))


def _tc_merge(Wg, Kg):
    bb = 16
    n3, n5, n7 = _L - 2, _L - 4, _L - 6

    def body(wg_ref, kg_ref, o3_ref, o5_ref, o7_ref):
        for b in range(bb):
            def prod(d, clo, n):
                w = wg_ref[b, pl.ds(clo + d, n), pl.ds(0, _EMB)]
                k = kg_ref[b, pl.ds(clo, n), pl.ds(_EMB * (3 + d), _EMB)]
                return w * k

            m = prod(-1, 1, n3)
            m = jnp.maximum(m, prod(0, 1, n3))
            m = jnp.maximum(m, prod(1, 1, n3))
            o3_ref[b] = m
            m = m[1:1 + n5]
            m = jnp.maximum(m, prod(-2, 2, n5))
            m = jnp.maximum(m, prod(2, 2, n5))
            o5_ref[b] = m
            m = m[1:1 + n7]
            m = jnp.maximum(m, prod(-3, 3, n7))
            m = jnp.maximum(m, prod(3, 3, n7))
            o7_ref[b] = m

    out = pl.pallas_call(
        body,
        grid=(_B // bb,),
        in_specs=[
            pl.BlockSpec((bb, _L, _WPAD), lambda i: (i, 0, 0)),
            pl.BlockSpec((bb, _L, _KPAD), lambda i: (i, 0, 0)),
        ],
        out_specs=[
            pl.BlockSpec((bb, n3, _EMB), lambda i: (i, 0, 0)),
            pl.BlockSpec((bb, n5, _EMB), lambda i: (i, 0, 0)),
            pl.BlockSpec((bb, n7, _EMB), lambda i: (i, 0, 0)),
        ],
        out_shape=[
            jax.ShapeDtypeStruct((_B, n3, _EMB), jnp.float32),
            jax.ShapeDtypeStruct((_B, n5, _EMB), jnp.float32),
            jax.ShapeDtypeStruct((_B, n7, _EMB), jnp.float32),
        ],
    )(Wg, Kg)
    return tuple(out)


@jax.jit
def kernel(seq, W, K):
    seq_flat = seq.astype(jnp.int32).reshape(-1)
    Wp = jnp.concatenate([W, W], axis=1)
    Wg = _sc_gather(seq_flat, Wp, _WPAD, 128)
    KT = jnp.transpose(K, (1, 2, 0)).reshape(_RMAX * _EMB, _VOCAB)
    Kp = _tc_pack_k(KT)
    Kg = _sc_gather(seq_flat, Kp, _KPAD, 64)
    Wg = Wg.reshape(_B, _L, _WPAD)
    Kg = Kg.reshape(_B, _L, _KPAD)
    return _tc_merge(Wg, Kg)
